# Initial kernel scaffold; baseline (speedup 1.0000x reference)
import functools
import jax
import jax.numpy as jnp
from jax import lax
from jax.experimental import pallas as pl
from jax.experimental.pallas import tpu as pltpu
from jax.experimental.pallas import tpu_sc as plsc

N = 10000
E = 320000
IN_DIM = 128
HID = 64
OUT_DIM = 128
HEADS = 8
NUM_PAR = 32

NC = 2
NS = 16
CE = 128
EP = 323584
EPT2 = EP // NS
NCH2 = EPT2 // CE
EPT4 = EP // (NC * NS)
NCH4 = EPT4 // CE
NP = N + 16
NB = 1000
GRID = N // NB

f32 = jnp.float32
i32 = jnp.int32


def _k1_body(h_ref, w1_ref, a1_ref, z_ref, e1_ref):
    z = jnp.dot(h_ref[...], w1_ref[...], preferred_element_type=f32)
    e1_ref[...] = lax.dot_general(a1_ref[...], z, (((0,), (1,)), ((), ())))
    for hd in range(HEADS):
        z_ref[hd] = z[:, hd * HID:(hd + 1) * HID]


def _k1(h, W1f, A1):
    return pl.pallas_call(
        _k1_body,
        grid=(GRID,),
        in_specs=[
            pl.BlockSpec((NB, IN_DIM), lambda i: (i, 0)),
            pl.BlockSpec((IN_DIM, HEADS * HID), lambda i: (0, 0)),
            pl.BlockSpec((HEADS * HID, 2 * HEADS), lambda i: (0, 0)),
        ],
        out_specs=[
            pl.BlockSpec((HEADS, NB, HID), lambda i: (0, i, 0)),
            pl.BlockSpec((2 * HEADS, NB), lambda i: (0, i)),
        ],
        out_shape=[
            jax.ShapeDtypeStruct((HEADS, N, HID), f32),
            jax.ShapeDtypeStruct((2 * HEADS, N), f32),
        ],
    )(h, W1f, A1)


def _k2_body(src_r, dst_r, e1_r, z_r, zacc_r, zden_r, acc_r, den_r,
             srcv, dstv, esv, edv, exv, idxv, zb, acc_sh, den_sh):
    c = lax.axis_index("c")
    s = lax.axis_index("s")
    pltpu.sync_copy(src_r.at[s], srcv)
    pltpu.sync_copy(dst_r.at[s], dstv)
    r0 = s * (NP // NS)

    for hi in range(HEADS // NC):
        h = c * (HEADS // NC) + hi
        pltpu.sync_copy(e1_r.at[h], esv.at[pl.ds(0, N)])
        pltpu.sync_copy(e1_r.at[HEADS + h], edv.at[pl.ds(0, N)])
        esv[pl.ds(N, 16)] = jnp.zeros((16,), f32)
        edv[pl.ds(N, 16)] = jnp.zeros((16,), f32)
        pltpu.sync_copy(zacc_r.at[pl.ds(r0, NP // NS)],
                        acc_sh.at[pl.ds(r0, NP // NS)])
        pltpu.sync_copy(zden_r.at[pl.ds(r0, NP // NS)],
                        den_sh.at[pl.ds(r0, NP // NS)])
        plsc.subcore_barrier()

        hN = h * N

        def chunk(g, carry):
            for j in range(CE // 16):
                sv = srcv[g, pl.ds(j * 16, 16)]
                dv = dstv[g, pl.ds(j * 16, 16)]
                e = plsc.load_gather(esv, [sv]) + plsc.load_gather(edv, [dv])
                e = jnp.where(e >= 0, e, 0.2 * e)
                exv[pl.ds(j * 16, 16)] = jnp.exp(e)
                idxv[pl.ds(j * 16, 16)] = sv + hN
            pltpu.sync_copy(z_r.at[idxv], zb)

            def srow(r, carry2):
                sc = exv[r]
                for k in range(HID // 16):
                    zb[r, pl.ds(k * 16, 16)] = zb[r, pl.ds(k * 16, 16)] * sc
                return carry2
            lax.fori_loop(0, CE, srow, 0)
            pltpu.sync_copy(zb, acc_sh.at[dstv.at[g]], add=True)
            pltpu.sync_copy(exv, den_sh.at[dstv.at[g]], add=True)
            return carry
        lax.fori_loop(0, NCH2, chunk, 0)
        plsc.subcore_barrier()
        pltpu.sync_copy(acc_sh.at[pl.ds(s * (N // NS), N // NS)],
                        acc_r.at[h, pl.ds(s * (N // NS), N // NS)])
        pltpu.sync_copy(den_sh.at[pl.ds(s * (N // NS), N // NS)],
                        den_r.at[h, pl.ds(s * (N // NS), N // NS)])
        plsc.subcore_barrier()


_k2 = functools.partial(
    pl.kernel,
    out_type=[
        jax.ShapeDtypeStruct((HEADS, N, HID), f32),
        jax.ShapeDtypeStruct((HEADS, N), f32),
    ],
    mesh=plsc.VectorSubcoreMesh(core_axis_name="c", subcore_axis_name="s"),
    scratch_types=[
        pltpu.VMEM((NCH2, CE), i32),
        pltpu.VMEM((NCH2, CE), i32),
        pltpu.VMEM((NP,), f32),
        pltpu.VMEM((NP,), f32),
        pltpu.VMEM((CE,), f32),
        pltpu.VMEM((CE,), i32),
        pltpu.VMEM((CE, HID), f32),
        pltpu.VMEM_SHARED((NP, HID), f32),
        pltpu.VMEM_SHARED((NP,), f32),
    ],
)(_k2_body)


def _k3_body(acc_r, den_r, w2_r, wss_r, a2_r, z2_r, e2_r, hss_r):
    z2 = jnp.zeros((NB, OUT_DIM), f32)
    hss = jnp.zeros((NB, NUM_PAR), f32)
    for hd in range(HEADS):
        x = acc_r[hd] / (den_r[hd][:, None] + 1e-9)
        hm = jnp.where(x > 0, x, jnp.expm1(x))
        z2 += jnp.dot(hm, w2_r[hd * HID:(hd + 1) * HID, :],
                      preferred_element_type=f32)
        hss += jnp.dot(hm, wss_r[hd * HID:(hd + 1) * HID, :],
                       preferred_element_type=f32)
    z2_r[...] = z2
    hss_r[...] = hss
    e2_r[...] = lax.dot_general(a2_r[...], z2, (((0,), (1,)), ((), ())))


def _k3(acc1, den1, W2f, Wss, A2):
    return pl.pallas_call(
        _k3_body,
        grid=(GRID,),
        in_specs=[
            pl.BlockSpec((HEADS, NB, HID), lambda i: (0, i, 0)),
            pl.BlockSpec((HEADS, NB), lambda i: (0, i)),
            pl.BlockSpec((HEADS * HID, OUT_DIM), lambda i: (0, 0)),
            pl.BlockSpec((HEADS * HID, NUM_PAR), lambda i: (0, 0)),
            pl.BlockSpec((OUT_DIM, 8), lambda i: (0, 0)),
        ],
        out_specs=[
            pl.BlockSpec((NB, OUT_DIM), lambda i: (i, 0)),
            pl.BlockSpec((8, NB), lambda i: (0, i)),
            pl.BlockSpec((NB, NUM_PAR), lambda i: (i, 0)),
        ],
        out_shape=[
            jax.ShapeDtypeStruct((N, OUT_DIM), f32),
            jax.ShapeDtypeStruct((8, N), f32),
            jax.ShapeDtypeStruct((N, NUM_PAR), f32),
        ],
    )(acc1, den1, W2f, Wss, A2)


def _k4_body(src_r, dst_r, e2_r, z_r, zacc_r, zden_r, acc_r, den_r,
             srcv, dstv, esv, edv, exv, zb, acc_sh, den_sh):
    c = lax.axis_index("c")
    s = lax.axis_index("s")
    w = c * NS + s
    pltpu.sync_copy(src_r.at[w], srcv)
    pltpu.sync_copy(dst_r.at[w], dstv)
    pltpu.sync_copy(e2_r.at[0], esv.at[pl.ds(0, N)])
    pltpu.sync_copy(e2_r.at[1], edv.at[pl.ds(0, N)])
    esv[pl.ds(N, 16)] = jnp.zeros((16,), f32)
    edv[pl.ds(N, 16)] = jnp.zeros((16,), f32)
    r0 = s * (NP // NS)
    pltpu.sync_copy(zacc_r.at[pl.ds(r0, NP // NS)],
                    acc_sh.at[pl.ds(r0, NP // NS)])
    pltpu.sync_copy(zden_r.at[pl.ds(r0, NP // NS)],
                    den_sh.at[pl.ds(r0, NP // NS)])
    plsc.subcore_barrier()

    def chunk(g, carry):
        for j in range(CE // 16):
            sv = srcv[g, pl.ds(j * 16, 16)]
            dv = dstv[g, pl.ds(j * 16, 16)]
            e = plsc.load_gather(esv, [sv]) + plsc.load_gather(edv, [dv])
            e = jnp.where(e >= 0, e, 0.2 * e)
            exv[pl.ds(j * 16, 16)] = jnp.exp(e)
        pltpu.sync_copy(z_r.at[srcv.at[g]], zb)

        def srow(r, carry2):
            sc = exv[r]
            for k in range(OUT_DIM // 16):
                zb[r, pl.ds(k * 16, 16)] = zb[r, pl.ds(k * 16, 16)] * sc
            return carry2
        lax.fori_loop(0, CE, srow, 0)
        pltpu.sync_copy(zb, acc_sh.at[dstv.at[g]], add=True)
        pltpu.sync_copy(exv, den_sh.at[dstv.at[g]], add=True)
        return carry
    lax.fori_loop(0, NCH4, chunk, 0)
    plsc.subcore_barrier()
    pltpu.sync_copy(acc_sh.at[pl.ds(s * (N // NS), N // NS)],
                    acc_r.at[c, pl.ds(s * (N // NS), N // NS)])
    pltpu.sync_copy(den_sh.at[pl.ds(s * (N // NS), N // NS)],
                    den_r.at[c, pl.ds(s * (N // NS), N // NS)])


_k4 = functools.partial(
    pl.kernel,
    out_type=[
        jax.ShapeDtypeStruct((NC, N, OUT_DIM), f32),
        jax.ShapeDtypeStruct((NC, N), f32),
    ],
    mesh=plsc.VectorSubcoreMesh(core_axis_name="c", subcore_axis_name="s"),
    scratch_types=[
        pltpu.VMEM((NCH4, CE), i32),
        pltpu.VMEM((NCH4, CE), i32),
        pltpu.VMEM((NP,), f32),
        pltpu.VMEM((NP,), f32),
        pltpu.VMEM((CE,), f32),
        pltpu.VMEM((CE, OUT_DIM), f32),
        pltpu.VMEM_SHARED((NP, OUT_DIM), f32),
        pltpu.VMEM_SHARED((NP,), f32),
    ],
)(_k4_body)


def _k5_body(acc_r, den_r, h2_r):
    x = (acc_r[0] + acc_r[1]) / (den_r[0][:, None] + den_r[1][:, None] + 1e-9)
    h2_r[...] = jnp.where(x > 0, x, jnp.expm1(x))


def _k5(acc2, den2):
    return pl.pallas_call(
        _k5_body,
        grid=(GRID,),
        in_specs=[
            pl.BlockSpec((NC, NB, OUT_DIM), lambda i: (0, i, 0)),
            pl.BlockSpec((NC, NB), lambda i: (0, i)),
        ],
        out_specs=pl.BlockSpec((NB, OUT_DIM), lambda i: (i, 0)),
        out_shape=jax.ShapeDtypeStruct((N, OUT_DIM), f32),
    )(acc2, den2)


def kernel(h, edge_index, snorm_n, snorm_e, W1, a1_src, a1_dst,
           W2, a2_src, a2_dst, W_ss):
    src = edge_index[0]
    dst = edge_index[1]
    pad = EP - E
    srcp = jnp.concatenate([src, jnp.zeros((pad,), i32)])
    dstp = jnp.concatenate([dst, jnp.full((pad,), N, i32)])
    src2 = srcp.reshape(NS, NCH2, CE)
    dst2 = dstp.reshape(NS, NCH2, CE)
    src4 = srcp.reshape(NC * NS, NCH4, CE)
    dst4 = dstp.reshape(NC * NS, NCH4, CE)

    W1f = W1.reshape(IN_DIM, HEADS * HID)
    eye_rep = jnp.repeat(jnp.eye(HEADS, dtype=f32), HID, axis=0)
    A1 = jnp.concatenate([eye_rep * a1_src.reshape(-1, 1),
                          eye_rep * a1_dst.reshape(-1, 1)], axis=1)
    W2f = W2.reshape(HEADS * HID, OUT_DIM)
    A2 = jnp.concatenate(
        [a2_src.T, a2_dst.T, jnp.zeros((OUT_DIM, 6), f32)], axis=1)

    zacc1 = jnp.zeros((NP, HID), f32)
    zacc2 = jnp.zeros((NP, OUT_DIM), f32)
    zden = jnp.zeros((NP,), f32)

    z1, e1 = _k1(h, W1f, A1)
    acc1, den1 = _k2(src2, dst2, e1, z1.reshape(HEADS * N, HID), zacc1, zden)
    z2, e2, hss = _k3(acc1, den1, W2f, W_ss, A2)
    acc2, den2 = _k4(src4, dst4, e2, z2, zacc2, zden)
    h2 = _k5(acc2, den2)
    return (h2, hss)


# speedup vs baseline: 24.3596x; 24.3596x over previous
import functools
import jax
import jax.numpy as jnp
from jax import lax
from jax.experimental import pallas as pl
from jax.experimental.pallas import tpu as pltpu
from jax.experimental.pallas import tpu_sc as plsc

N = 10000
E = 320000
IN_DIM = 128
HID = 64
OUT_DIM = 128
HEADS = 8
NUM_PAR = 32

NC = 2
NS = 16
CE = 128
EP = 323584
EPT2 = EP // NS
NCH2 = EPT2 // CE
EPT4 = EP // (NC * NS)
NCH4 = EPT4 // CE
NQ = 10240
NQT = NQ // NS
NB = 1024
GRID = NQ // NB

f32 = jnp.float32
i32 = jnp.int32


def _k1_body(h_ref, w1_ref, a1_ref, z_ref, e1_ref):
    z = jnp.dot(h_ref[...], w1_ref[...], preferred_element_type=f32)
    e1_ref[...] = lax.dot_general(a1_ref[...], z, (((0,), (1,)), ((), ())))
    for hd in range(HEADS):
        z_ref[hd] = z[:, hd * HID:(hd + 1) * HID]


def _k1(h, W1f, A1):
    return pl.pallas_call(
        _k1_body,
        grid=(GRID,),
        in_specs=[
            pl.BlockSpec((NB, IN_DIM), lambda i: (i, 0)),
            pl.BlockSpec((IN_DIM, HEADS * HID), lambda i: (0, 0)),
            pl.BlockSpec((HEADS * HID, 2 * HEADS), lambda i: (0, 0)),
        ],
        out_specs=[
            pl.BlockSpec((HEADS, NB, HID), lambda i: (0, i, 0)),
            pl.BlockSpec((2 * HEADS, NB), lambda i: (0, i)),
        ],
        out_shape=[
            jax.ShapeDtypeStruct((HEADS, NQ, HID), f32),
            jax.ShapeDtypeStruct((2 * HEADS, NQ), f32),
        ],
    )(h, W1f, A1)


def _k2_body(src_r, dst_r, e1_r, z_r, zacc_r, zden_r, acc_r, den_r,
             srcv, dstv, esv, edv, exv, idxv, zb, acc_sh, den_sh):
    c = lax.axis_index("c")
    s = lax.axis_index("s")
    pltpu.sync_copy(src_r.at[s], srcv)
    pltpu.sync_copy(dst_r.at[s], dstv)
    r0 = s * NQT

    for hi in range(HEADS // NC):
        h = c * (HEADS // NC) + hi
        pltpu.sync_copy(e1_r.at[h], esv)
        pltpu.sync_copy(e1_r.at[HEADS + h], edv)
        pltpu.sync_copy(zacc_r.at[pl.ds(r0, NQT)], acc_sh.at[pl.ds(r0, NQT)])
        pltpu.sync_copy(zden_r.at[pl.ds(r0, NQT)], den_sh.at[pl.ds(r0, NQT)])
        plsc.subcore_barrier()

        hN = h * NQ

        def chunk(g, carry):
            exs = []
            for j in range(CE // 16):
                sv = srcv[g, pl.ds(j * 16, 16)]
                dv = dstv[g, pl.ds(j * 16, 16)]
                e = plsc.load_gather(esv, [sv]) + plsc.load_gather(edv, [dv])
                e = jnp.where(e >= 0, e, 0.2 * e)
                ex = jnp.exp(e)
                exs.append(ex)
                exv[pl.ds(j * 16, 16)] = ex
                idxv[pl.ds(j * 16, 16)] = sv + hN
            pltpu.sync_copy(z_r.at[idxv], zb)
            for j in range(CE // 16):
                for rr in range(16):
                    sc = exs[j][rr]
                    r = j * 16 + rr
                    for k in range(HID // 16):
                        zb[r, pl.ds(k * 16, 16)] = (
                            zb[r, pl.ds(k * 16, 16)] * sc)
            pltpu.sync_copy(zb, acc_sh.at[dstv.at[g]], add=True)
            pltpu.sync_copy(exv, den_sh.at[dstv.at[g]], add=True)
            return carry
        lax.fori_loop(0, NCH2, chunk, 0)
        plsc.subcore_barrier()
        pltpu.sync_copy(acc_sh.at[pl.ds(r0, NQT)],
                        acc_r.at[h, pl.ds(r0, NQT)])
        pltpu.sync_copy(den_sh.at[pl.ds(r0, NQT)],
                        den_r.at[h, pl.ds(r0, NQT)])
        plsc.subcore_barrier()


@functools.cache
def _k2():
    return pl.kernel(
        _k2_body,
        out_type=[
            jax.ShapeDtypeStruct((HEADS, NQ, HID), f32),
            jax.ShapeDtypeStruct((HEADS, NQ), f32),
        ],
        mesh=plsc.VectorSubcoreMesh(core_axis_name="c", subcore_axis_name="s",
                                    num_cores=NC, num_subcores=NS),
        compiler_params=pltpu.CompilerParams(
            use_tc_tiling_on_sc=False, needs_layout_passes=False),
        scratch_types=[
            pltpu.VMEM((NCH2, CE), i32),
            pltpu.VMEM((NCH2, CE), i32),
            pltpu.VMEM((NQ,), f32),
            pltpu.VMEM((NQ,), f32),
            pltpu.VMEM((CE,), f32),
            pltpu.VMEM((CE,), i32),
            pltpu.VMEM((CE, HID), f32),
            pltpu.VMEM_SHARED((NQ, HID), f32),
            pltpu.VMEM_SHARED((NQ,), f32),
        ],
    )


def _k3_body(acc_r, den_r, w2_r, wss_r, a2_r, z2_r, e2_r, hss_r):
    z2 = jnp.zeros((NB, OUT_DIM), f32)
    hss = jnp.zeros((NB, NUM_PAR), f32)
    for hd in range(HEADS):
        x = acc_r[hd] / (den_r[hd][:, None] + 1e-9)
        hm = jnp.where(x > 0, x, (jnp.exp(x) - 1.0))
        z2 += jnp.dot(hm, w2_r[hd * HID:(hd + 1) * HID, :],
                      preferred_element_type=f32)
        hss += jnp.dot(hm, wss_r[hd * HID:(hd + 1) * HID, :],
                       preferred_element_type=f32)
    z2_r[...] = z2
    hss_r[...] = hss
    e2_r[...] = lax.dot_general(a2_r[...], z2, (((0,), (1,)), ((), ())))


def _k3(acc1, den1, W2f, Wss, A2):
    return pl.pallas_call(
        _k3_body,
        grid=(GRID,),
        in_specs=[
            pl.BlockSpec((HEADS, NB, HID), lambda i: (0, i, 0)),
            pl.BlockSpec((HEADS, NB), lambda i: (0, i)),
            pl.BlockSpec((HEADS * HID, OUT_DIM), lambda i: (0, 0)),
            pl.BlockSpec((HEADS * HID, NUM_PAR), lambda i: (0, 0)),
            pl.BlockSpec((OUT_DIM, 8), lambda i: (0, 0)),
        ],
        out_specs=[
            pl.BlockSpec((NB, OUT_DIM), lambda i: (i, 0)),
            pl.BlockSpec((8, NB), lambda i: (0, i)),
            pl.BlockSpec((NB, NUM_PAR), lambda i: (i, 0)),
        ],
        out_shape=[
            jax.ShapeDtypeStruct((NQ, OUT_DIM), f32),
            jax.ShapeDtypeStruct((8, NQ), f32),
            jax.ShapeDtypeStruct((NQ, NUM_PAR), f32),
        ],
    )(acc1, den1, W2f, Wss, A2)


def _k4_body(src_r, dst_r, e2_r, z_r, zacc_r, zden_r, acc_r, den_r,
             srcv, dstv, esv, edv, exv, zb, acc_sh, den_sh):
    c = lax.axis_index("c")
    s = lax.axis_index("s")
    w = c * NS + s
    pltpu.sync_copy(e2_r.at[0], esv)
    pltpu.sync_copy(e2_r.at[1], edv)
    r0 = s * NQT
    pltpu.sync_copy(zacc_r.at[pl.ds(r0, NQT)], acc_sh.at[pl.ds(r0, NQT)])
    pltpu.sync_copy(zden_r.at[pl.ds(r0, NQT)], den_sh.at[pl.ds(r0, NQT)])
    plsc.subcore_barrier()

    def chunk(g, carry):
        pltpu.sync_copy(src_r.at[w, g], srcv)
        pltpu.sync_copy(dst_r.at[w, g], dstv)
        exs = []
        for j in range(CE // 16):
            sv = srcv[pl.ds(j * 16, 16)]
            dv = dstv[pl.ds(j * 16, 16)]
            e = plsc.load_gather(esv, [sv]) + plsc.load_gather(edv, [dv])
            e = jnp.where(e >= 0, e, 0.2 * e)
            ex = jnp.exp(e)
            exs.append(ex)
            exv[pl.ds(j * 16, 16)] = ex
        pltpu.sync_copy(z_r.at[srcv], zb)
        for j in range(CE // 16):
            for rr in range(16):
                sc = exs[j][rr]
                r = j * 16 + rr
                for k in range(OUT_DIM // 16):
                    zb[r, pl.ds(k * 16, 16)] = zb[r, pl.ds(k * 16, 16)] * sc
        pltpu.sync_copy(zb, acc_sh.at[dstv], add=True)
        pltpu.sync_copy(exv, den_sh.at[dstv], add=True)
        return carry
    lax.fori_loop(0, NCH4, chunk, 0)
    plsc.subcore_barrier()
    pltpu.sync_copy(acc_sh.at[pl.ds(r0, NQT)], acc_r.at[c, pl.ds(r0, NQT)])
    pltpu.sync_copy(den_sh.at[pl.ds(r0, NQT)], den_r.at[c, pl.ds(r0, NQT)])


@functools.cache
def _k4():
    return pl.kernel(
        _k4_body,
        out_type=[
            jax.ShapeDtypeStruct((NC, NQ, OUT_DIM), f32),
            jax.ShapeDtypeStruct((NC, NQ), f32),
        ],
        mesh=plsc.VectorSubcoreMesh(core_axis_name="c", subcore_axis_name="s",
                                    num_cores=NC, num_subcores=NS),
        compiler_params=pltpu.CompilerParams(
            use_tc_tiling_on_sc=False, needs_layout_passes=False),
        scratch_types=[
            pltpu.VMEM((CE,), i32),
            pltpu.VMEM((CE,), i32),
            pltpu.VMEM((NQ,), f32),
            pltpu.VMEM((NQ,), f32),
            pltpu.VMEM((CE,), f32),
            pltpu.VMEM((CE, OUT_DIM), f32),
            pltpu.VMEM_SHARED((NQ, OUT_DIM), f32),
            pltpu.VMEM_SHARED((NQ,), f32),
        ],
    )


def _k5_body(acc_r, den_r, h2_r):
    x = (acc_r[0] + acc_r[1]) / (den_r[0][:, None] + den_r[1][:, None] + 1e-9)
    h2_r[...] = jnp.where(x > 0, x, (jnp.exp(x) - 1.0))


def _k5(acc2, den2):
    return pl.pallas_call(
        _k5_body,
        grid=(GRID,),
        in_specs=[
            pl.BlockSpec((NC, NB, OUT_DIM), lambda i: (0, i, 0)),
            pl.BlockSpec((NC, NB), lambda i: (0, i)),
        ],
        out_specs=pl.BlockSpec((NB, OUT_DIM), lambda i: (i, 0)),
        out_shape=jax.ShapeDtypeStruct((NQ, OUT_DIM), f32),
    )(acc2, den2)


def kernel(h, edge_index, snorm_n, snorm_e, W1, a1_src, a1_dst,
           W2, a2_src, a2_dst, W_ss):
    src = edge_index[0]
    dst = edge_index[1]
    pad = EP - E
    srcp = jnp.concatenate([src, jnp.zeros((pad,), i32)])
    dstp = jnp.concatenate([dst, jnp.full((pad,), N, i32)])
    src2 = srcp.reshape(NS, NCH2, CE)
    dst2 = dstp.reshape(NS, NCH2, CE)
    src4 = srcp.reshape(NC * NS, NCH4, CE)
    dst4 = dstp.reshape(NC * NS, NCH4, CE)

    hq = jnp.pad(h, ((0, NQ - N), (0, 0)))
    W1f = W1.reshape(IN_DIM, HEADS * HID)
    eye_rep = jnp.repeat(jnp.eye(HEADS, dtype=f32), HID, axis=0)
    A1 = jnp.concatenate([eye_rep * a1_src.reshape(-1, 1),
                          eye_rep * a1_dst.reshape(-1, 1)], axis=1)
    W2f = W2.reshape(HEADS * HID, OUT_DIM)
    A2 = jnp.concatenate(
        [a2_src.T, a2_dst.T, jnp.zeros((OUT_DIM, 6), f32)], axis=1)

    zacc1 = jnp.zeros((NQ, HID), f32)
    zacc2 = jnp.zeros((NQ, OUT_DIM), f32)
    zden = jnp.zeros((NQ,), f32)

    z1, e1 = _k1(hq, W1f, A1)
    acc1, den1 = _k2()(src2, dst2, e1, z1.reshape(HEADS * NQ, HID),
                       zacc1, zden)
    z2, e2, hss = _k3(acc1, den1, W2f, W_ss, A2)
    acc2, den2 = _k4()(src4, dst4, e2, z2, zacc2, zden)
    h2 = _k5(acc2, den2)
    return (h2[:N], hss[:N])
